# Initial kernel scaffold; baseline (speedup 1.0000x reference)
import jax
import jax.numpy as jnp
from jax.experimental import pallas as pl


def kernel(x, edge_index, edge_attr, enc_W, enc_b, enc_gamma, enc_beta, W, att_src, att_dst, W_edge, att_edge, bias, bn_gamma, bn_beta, out_W, out_b):
    raise NotImplementedError("write your pallas kernel here")


# speedup vs baseline: 1.7329x; 1.7329x over previous
import jax
import jax.numpy as jnp
from jax.experimental import pallas as pl
from jax.experimental.pallas import tpu as pltpu


def _encoder_body(x_ref, w_ref, b_ref, g_ref, be_ref, h_ref):
    y = jnp.dot(x_ref[...], w_ref[...], preferred_element_type=jnp.float32)
    y = y + b_ref[...][None, :]
    m = jnp.mean(y, axis=0)
    v = jnp.mean((y - m[None, :]) ** 2, axis=0)
    y = (y - m[None, :]) * jax.lax.rsqrt(v + 1e-5) * g_ref[...][None, :] + be_ref[...][None, :]
    h_ref[...] = jnp.maximum(y, 0.0)


def kernel(x, edge_index, edge_attr, enc_W, enc_b, enc_gamma, enc_beta, W,
           att_src, att_dst, W_edge, att_edge, bias, bn_gamma, bn_beta, out_W, out_b):
    n = x.shape[0]
    h = pl.pallas_call(
        _encoder_body,
        out_shape=jax.ShapeDtypeStruct((n, x.shape[1] and enc_W.shape[1]), jnp.float32),
    )(x, enc_W, enc_b, enc_gamma, enc_beta)

    src, dst = edge_index[0], edge_index[1]
    ones = jnp.ones((edge_attr.shape[0],), dtype=x.dtype)
    deg = jax.ops.segment_sum(ones, dst, num_segments=n)
    loop_attr = jax.ops.segment_sum(edge_attr, dst, num_segments=n) / jnp.clip(deg, 1.0)[:, None]
    loop = jnp.arange(n, dtype=src.dtype)
    src2 = jnp.concatenate([src, loop])
    dst2 = jnp.concatenate([dst, loop])
    ea2 = jnp.concatenate([edge_attr, loop_attr], axis=0)

    L = W.shape[0]
    for l in range(L):
        xs = h @ W[l]
        a_s = xs @ att_src[l]
        a_d = xs @ att_dst[l]
        a_e = ea2 @ (W_edge[l] @ att_edge[l])
        alpha = a_s[src2] + a_d[dst2] + a_e
        alpha = jax.nn.leaky_relu(alpha, 0.2)
        s = jnp.exp(alpha)
        denom = jax.ops.segment_sum(s, dst2, num_segments=n)
        msg = xs[src2] * s[:, None]
        out = jax.ops.segment_sum(msg, dst2, num_segments=n)
        out = out / (denom[:, None] + 1e-16) + bias[l]
        m = jnp.mean(out, axis=0)
        v = jnp.var(out, axis=0)
        out = (out - m) / jnp.sqrt(v + 1e-5) * bn_gamma[l] + bn_beta[l]
        h = h + jnp.maximum(out, 0.0)
    g = jnp.mean(h, axis=0, keepdims=True)
    return jnp.maximum(g @ out_W + out_b, 0.0)


# speedup vs baseline: 18.1810x; 10.4915x over previous
import functools

import jax
import jax.numpy as jnp
from jax import lax
from jax.experimental import pallas as pl
from jax.experimental.pallas import tpu as pltpu
from jax.experimental.pallas import tpu_sc as plsc

NC = 2
NS = 16
NW = NC * NS
CH = 128


def _ceil_div(a, b):
    return -(-a // b)


def _encoder_body(x_ref, w_ref, b_ref, g_ref, be_ref, h_ref):
    y = jnp.dot(x_ref[...], w_ref[...], preferred_element_type=jnp.float32)
    y = y + b_ref[...][None, :]
    m = jnp.mean(y, axis=0)
    v = jnp.mean((y - m[None, :]) ** 2, axis=0)
    y = (y - m[None, :]) * lax.rsqrt(v + 1e-5) * g_ref[...][None, :] + be_ref[...][None, :]
    h_ref[...] = jnp.maximum(y, 0.0)


def _loop_stats_sc(dst, edge_attr, n_pad):
    E = dst.shape[0]
    T = _ceil_div(_ceil_div(E, NW), CH)
    per_w = T * CH
    EP = per_w * NW
    dstp = jnp.zeros((EP,), jnp.int32).at[:E].set(dst).reshape(NW, T, CH)
    attrp = jnp.zeros((EP, 32), jnp.float32)
    attrp = attrp.at[:E, :16].set(edge_attr).at[:E, 16].set(1.0)
    attrp = attrp.reshape(NW, per_w, 32)
    stripe = n_pad // NS

    mesh = plsc.VectorSubcoreMesh(core_axis_name="c", subcore_axis_name="s")

    @functools.partial(
        pl.kernel,
        out_type=jax.ShapeDtypeStruct((NC, n_pad, 32), jnp.float32),
        mesh=mesh,
        compiler_params=pltpu.CompilerParams(needs_layout_passes=False, use_tc_tiling_on_sc=False),
        scratch_types=[
            pltpu.VMEM((T, CH), jnp.int32),
            pltpu.VMEM((CH, 32), jnp.float32),
            pltpu.VMEM((CH, 32), jnp.float32),
            pltpu.VMEM_SHARED((n_pad, 32), jnp.float32),
        ],
    )
    def k(dst_hbm, attr_hbm, ssum_hbm, dst_vm, attr_vm, zero_vm, s_sh):
        core = lax.axis_index("c")
        sub = lax.axis_index("s")
        wid = sub * NC + core
        zeros16 = jnp.zeros((16,), jnp.float32)
        pltpu.sync_copy(dst_hbm.at[wid], dst_vm)

        def z_zero(i, carry):
            zero_vm[i, pl.ds(0, 16)] = zeros16
            zero_vm[i, pl.ds(16, 16)] = zeros16
            return carry

        lax.fori_loop(0, CH, z_zero, 0)

        base = sub * stripe
        for off in range(0, stripe, CH):
            pltpu.sync_copy(zero_vm, s_sh.at[pl.ds(base + off, CH)])
        plsc.subcore_barrier()

        def chunk(c, carry):
            pltpu.sync_copy(attr_hbm.at[wid, pl.ds(c * CH, CH)], attr_vm)
            pltpu.sync_copy(attr_vm, s_sh.at[dst_vm.at[c]], add=True)
            return carry

        lax.fori_loop(0, T, chunk, 0)

        plsc.subcore_barrier()
        pltpu.sync_copy(s_sh.at[pl.ds(base, stripe)],
                        ssum_hbm.at[core, pl.ds(base, stripe)])

    return k(dstp, attrp)


def _edge_pass_sc(idxp, a_s, a_d, xs, n_pad, E2):
    n = a_s.shape[0]
    T = idxp.shape[1]
    per_w = T * CH
    stripe = n_pad // NS

    mesh = plsc.VectorSubcoreMesh(core_axis_name="c", subcore_axis_name="s")

    @functools.partial(
        pl.kernel,
        out_type=(jax.ShapeDtypeStruct((NC, n_pad, 128), jnp.float32),
                  jax.ShapeDtypeStruct((NC, n_pad), jnp.float32)),
        mesh=mesh,
        compiler_params=pltpu.CompilerParams(needs_layout_passes=False, use_tc_tiling_on_sc=False),
        scratch_types=[
            pltpu.VMEM((3, CH), jnp.int32),
            pltpu.VMEM((n,), jnp.float32),
            pltpu.VMEM((n,), jnp.float32),
            pltpu.VMEM((CH, 128), jnp.float32),
            pltpu.VMEM((CH + 16,), jnp.float32),
            pltpu.VMEM_SHARED((n_pad, 128), jnp.float32),
            pltpu.VMEM_SHARED((n_pad,), jnp.float32),
            pltpu.SemaphoreType.DMA,
        ],
    )
    def k(idx_hbm, as_hbm, ad_hbm, xs_hbm, m_hbm, den_hbm,
          idx_vm, as_vm, ad_vm, rows_vm, s_vm, m_sh, d_sh, sem):
        core = lax.axis_index("c")
        sub = lax.axis_index("s")
        wid = sub * NC + core
        zeros16 = jnp.zeros((16,), jnp.float32)
        pltpu.sync_copy(as_hbm, as_vm)
        pltpu.sync_copy(ad_hbm, ad_vm)

        def z_rows(i, carry):
            for j in range(8):
                rows_vm[i, pl.ds(j * 16, 16)] = zeros16
            return carry

        lax.fori_loop(0, CH, z_rows, 0)
        for j in range(9):
            s_vm[pl.ds(j * 16, 16)] = zeros16

        base = sub * stripe
        for off in range(0, stripe, CH):
            pltpu.sync_copy(rows_vm, m_sh.at[pl.ds(base + off, CH)])
            pltpu.sync_copy(s_vm.at[pl.ds(0, CH)], d_sh.at[pl.ds(base + off, CH)])
        plsc.subcore_barrier()

        def chunk(c, carry):
            pltpu.sync_copy(idx_hbm.at[wid, c], idx_vm)
            gather = pltpu.async_copy(xs_hbm.at[idx_vm.at[0]], rows_vm, sem)
            for v in range(8):
                sidx = idx_vm[0, pl.ds(v * 16, 16)]
                didx = idx_vm[1, pl.ds(v * 16, 16)]
                ae = plsc.bitcast(idx_vm[2, pl.ds(v * 16, 16)], jnp.float32)
                al = (plsc.load_gather(as_vm, [sidx])
                      + plsc.load_gather(ad_vm, [didx]) + ae)
                al = jnp.maximum(al, 0.2 * al)
                s = jnp.exp(al)
                gid = wid * per_w + c * CH + v * 16 + lax.iota(jnp.int32, 16)
                s = jnp.where(gid < E2, s, 0.0)
                s_vm[pl.ds(v * 16, 16)] = s
            gather.wait()

            def scale(i, carry):
                sc = s_vm[pl.ds(i, 16)][0]
                for j in range(8):
                    rows_vm[i, pl.ds(j * 16, 16)] = rows_vm[i, pl.ds(j * 16, 16)] * sc
                return carry

            lax.fori_loop(0, CH, scale, 0)
            pltpu.sync_copy(rows_vm, m_sh.at[idx_vm.at[1]], add=True)
            pltpu.sync_copy(s_vm.at[pl.ds(0, CH)], d_sh.at[idx_vm.at[1]], add=True)
            return carry

        lax.fori_loop(0, T, chunk, 0)

        plsc.subcore_barrier()
        pltpu.sync_copy(m_sh.at[pl.ds(base, stripe)],
                        m_hbm.at[core, pl.ds(base, stripe)])
        pltpu.sync_copy(d_sh.at[pl.ds(base, stripe)],
                        den_hbm.at[core, pl.ds(base, stripe)])

    return k(idxp, a_s, a_d, xs)


def _dense_body(h_ref, w_ref, asrc_ref, adst_ref, xs_ref, as_ref, ad_ref):
    xs = jnp.dot(h_ref[...], w_ref[...], preferred_element_type=jnp.float32)
    xs_ref[...] = xs
    as_ref[...] = xs @ asrc_ref[...]
    ad_ref[...] = xs @ adst_ref[...]


def _post_body(m_ref, den_ref, bias_ref, bng_ref, bnb_ref, h_ref, hout_ref):
    n = h_ref.shape[0]
    msum = m_ref[0] + m_ref[1]
    den = den_ref[0] + den_ref[1]
    out = msum[:n] / (den[:n, None] + 1e-16) + bias_ref[...][None, :]
    m = jnp.mean(out, axis=0)
    v = jnp.mean((out - m[None, :]) ** 2, axis=0)
    out = (out - m[None, :]) * lax.rsqrt(v + 1e-5) * bng_ref[...][None, :] + bnb_ref[...][None, :]
    hout_ref[...] = h_ref[...] + jnp.maximum(out, 0.0)


def _head_body(h_ref, w_ref, b_ref, o_ref):
    g = jnp.mean(h_ref[...], axis=0, keepdims=True)
    o_ref[...] = jnp.maximum(
        jnp.dot(g, w_ref[...], preferred_element_type=jnp.float32) + b_ref[...][None, :],
        0.0)


def _ae_real_body(attr_r_ref, we_ref, ae_ref, out_ref):
    L = we_ref.shape[0]
    w_all = jnp.sum(we_ref[...] * ae_ref[...][:, None, :], axis=2)
    eye8 = (lax.broadcasted_iota(jnp.int32, (8, 8), 0) ==
            lax.broadcasted_iota(jnp.int32, (8, 8), 1)).astype(jnp.float32)
    wbig = (eye8[:, None, :, None] *
            jnp.transpose(w_all)[None, :, None, :]).reshape(128, 8 * L)
    out_ref[...] = jnp.dot(attr_r_ref[...], wbig,
                           preferred_element_type=jnp.float32)


def _loop_ae_body(ssum_ref, we_ref, ae_ref, loopae_ref):
    w_all = jnp.sum(we_ref[...] * ae_ref[...][:, None, :], axis=2)
    n = loopae_ref.shape[0]
    ssum = ssum_ref[0] + ssum_ref[1]
    attr_sum = ssum[:n, :16]
    deg = ssum[:n, 16]
    inv = 1.0 / jnp.maximum(deg, 1.0)
    loopae_ref[...] = jnp.dot(attr_sum, jnp.transpose(w_all),
                              preferred_element_type=jnp.float32) * inv[:, None]


def kernel(x, edge_index, edge_attr, enc_W, enc_b, enc_gamma, enc_beta, W,
           att_src, att_dst, W_edge, att_edge, bias, bn_gamma, bn_beta, out_W, out_b):
    n = x.shape[0]
    E = edge_index.shape[1]
    L = W.shape[0]
    src, dst = edge_index[0], edge_index[1]

    h = pl.pallas_call(
        _encoder_body,
        out_shape=jax.ShapeDtypeStruct((n, enc_W.shape[1]), jnp.float32),
    )(x, enc_W, enc_b, enc_gamma, enc_beta)

    n_pad = NS * CH * _ceil_div(n, NS * CH)
    ssum = _loop_stats_sc(dst, edge_attr, n_pad)

    attr_r = edge_attr.reshape(E // 8, 128)
    n_blk = 8
    br = E // 8 // n_blk
    a_real = pl.pallas_call(
        _ae_real_body,
        grid=(n_blk,),
        in_specs=[pl.BlockSpec((br, 128), lambda i: (i, 0)),
                  pl.BlockSpec(W_edge.shape, lambda i: (0, 0, 0)),
                  pl.BlockSpec(att_edge.shape, lambda i: (0, 0))],
        out_specs=pl.BlockSpec((br, 8 * L), lambda i: (i, 0)),
        out_shape=jax.ShapeDtypeStruct((E // 8, 8 * L), jnp.float32),
    )(attr_r, W_edge, att_edge).reshape(E, L)
    loop_ae = pl.pallas_call(
        _loop_ae_body,
        out_shape=jax.ShapeDtypeStruct((n, L), jnp.float32),
    )(ssum, W_edge, att_edge)

    loop = jnp.arange(n, dtype=src.dtype)
    src2 = jnp.concatenate([src, loop])
    dst2 = jnp.concatenate([dst, loop])
    ae2 = jnp.concatenate([a_real, loop_ae], axis=0)

    E2 = E + n
    T = _ceil_div(_ceil_div(E2, NW), CH)
    per_w = T * CH
    E2P = per_w * NW
    srcp = jnp.zeros((E2P,), jnp.int32).at[:E2].set(src2).reshape(NW, T, CH)
    dstp = jnp.zeros((E2P,), jnp.int32).at[:E2].set(dst2).reshape(NW, T, CH)
    aep = jnp.zeros((E2P, L), jnp.float32).at[:E2].set(ae2)
    ae_bits = lax.bitcast_convert_type(aep, jnp.int32).reshape(NW, T, CH, L)
    idxp = [jnp.stack([srcp, dstp, ae_bits[..., l]], axis=2) for l in range(L)]

    for l in range(L):
        xs, a_s, a_d = pl.pallas_call(
            _dense_body,
            out_shape=(jax.ShapeDtypeStruct((n, W.shape[2]), jnp.float32),
                       jax.ShapeDtypeStruct((n,), jnp.float32),
                       jax.ShapeDtypeStruct((n,), jnp.float32)),
        )(h, W[l], att_src[l], att_dst[l])
        m_parts, den_parts = _edge_pass_sc(idxp[l], a_s, a_d, xs, n_pad, E2)
        h = pl.pallas_call(
            _post_body,
            out_shape=jax.ShapeDtypeStruct((n, W.shape[2]), jnp.float32),
        )(m_parts, den_parts, bias[l], bn_gamma[l], bn_beta[l], h)

    return pl.pallas_call(
        _head_body,
        out_shape=jax.ShapeDtypeStruct((1, out_W.shape[1]), jnp.float32),
    )(h, out_W, out_b)
